# baseline (speedup 1.0000x reference)
import functools

import jax
import jax.numpy as jnp
from jax import lax
from jax.experimental import pallas as pl
from jax.experimental.pallas import tpu as pltpu

_INT_MAX = jnp.iinfo(jnp.int32).max


def _dist_body(qm2_ref, keys_ref, omin_ref, oidx_ref, *, nb):
    b = pl.program_id(0)
    kblk = keys_ref[...]
    qm2 = qm2_ref[...]
    dn = (((1,), (1,)), ((), ()))
    s = lax.dot_general(qm2, kblk, dn,
                        preferred_element_type=jnp.float32,
                        precision=lax.Precision.HIGHEST)
    k2 = lax.dot_general(jnp.ones_like(qm2), kblk * kblk, dn,
                         preferred_element_type=jnp.float32,
                         precision=lax.Precision.HIGHEST)
    dist = s + k2
    bmin = jnp.min(dist, axis=1, keepdims=True)
    ids = lax.broadcasted_iota(jnp.int32, dist.shape, 1) + b * nb
    cand = jnp.where(dist == bmin, ids, _INT_MAX)
    bidx = jnp.min(cand, axis=1, keepdims=True)
    omin_ref[...] = bmin[None]
    oidx_ref[...] = bidx[None]


def _nn_blockmin(queries, keys, nb, interpret=False):
    q, d = queries.shape
    k = keys.shape[0]
    assert k % nb == 0
    nblk = k // nb
    qm2 = -2.0 * queries
    omin, oidx = pl.pallas_call(
        functools.partial(_dist_body, nb=nb),
        grid=(nblk,),
        in_specs=[pl.BlockSpec((q, d), lambda b: (0, 0)),
                  pl.BlockSpec((nb, d), lambda b: (b, 0))],
        out_specs=[pl.BlockSpec((1, q, 1), lambda b: (b, 0, 0)),
                   pl.BlockSpec((1, q, 1), lambda b: (b, 0, 0))],
        out_shape=[jax.ShapeDtypeStruct((nblk, q, 1), jnp.float32),
                   jax.ShapeDtypeStruct((nblk, q, 1), jnp.int32)],
        interpret=interpret,
    )(qm2, keys)
    return omin[:, :, 0], oidx[:, :, 0]


def kernel(queries, keys, values):
    q = queries.shape[0]
    nb = 20000 if keys.shape[0] % 20000 == 0 else keys.shape[0]
    omin, oidx = _nn_blockmin(queries, keys, nb)
    bb = jnp.argmin(omin, axis=0)
    best_index = oidx[bb, jnp.arange(q)]
    return jnp.take(values, best_index, axis=0)


# speedup vs baseline: 1.5276x; 1.5276x over previous
import functools

import jax
import jax.numpy as jnp
from jax import lax
from jax.experimental import pallas as pl
from jax.experimental.pallas import tpu as pltpu

_INT_MAX = jnp.iinfo(jnp.int32).max


def _dist_body(qaug_ref, keys_ref, omin_ref, oidx_ref, *, nb):
    b = pl.program_id(0)
    kblk = keys_ref[...]
    qaug = qaug_ref[...]
    dn = (((1,), (1,)), ((), ()))
    kaug = jnp.concatenate([kblk, kblk * kblk], axis=1)
    dist = lax.dot_general(qaug, kaug, dn,
                           preferred_element_type=jnp.float32,
                           precision=lax.Precision.HIGHEST)
    bmin = jnp.min(dist, axis=1, keepdims=True)
    ids = lax.broadcasted_iota(jnp.int32, dist.shape, 1) + b * nb
    cand = jnp.where(dist == bmin, ids, _INT_MAX)
    bidx = jnp.min(cand, axis=1, keepdims=True)
    omin_ref[...] = bmin[None]
    oidx_ref[...] = bidx[None]


def _nn_blockmin(queries, keys, nb, interpret=False):
    q, d = queries.shape
    k = keys.shape[0]
    assert k % nb == 0
    nblk = k // nb
    qaug = jnp.concatenate([-2.0 * queries, jnp.ones_like(queries)], axis=1)
    omin, oidx = pl.pallas_call(
        functools.partial(_dist_body, nb=nb),
        grid=(nblk,),
        in_specs=[pl.BlockSpec((q, 2 * d), lambda b: (0, 0)),
                  pl.BlockSpec((nb, d), lambda b: (b, 0))],
        out_specs=[pl.BlockSpec((1, q, 1), lambda b: (b, 0, 0)),
                   pl.BlockSpec((1, q, 1), lambda b: (b, 0, 0))],
        out_shape=[jax.ShapeDtypeStruct((nblk, q, 1), jnp.float32),
                   jax.ShapeDtypeStruct((nblk, q, 1), jnp.int32)],
        interpret=interpret,
    )(qaug, keys)
    return omin[:, :, 0], oidx[:, :, 0]


def kernel(queries, keys, values):
    q = queries.shape[0]
    nb = 20000 if keys.shape[0] % 20000 == 0 else keys.shape[0]
    omin, oidx = _nn_blockmin(queries, keys, nb)
    bb = jnp.argmin(omin, axis=0)
    best_index = oidx[bb, jnp.arange(q)]
    return jnp.take(values, best_index, axis=0)


# speedup vs baseline: 1.5426x; 1.0098x over previous
import functools

import jax
import jax.numpy as jnp
from jax import lax
from jax.experimental import pallas as pl
from jax.experimental.pallas import tpu as pltpu

_INT_MAX = jnp.iinfo(jnp.int32).max


def _dist_body(qaug_ref, keys_ref, omin_ref, oidx_ref, *, nb):
    b = pl.program_id(0)
    kblk = keys_ref[...]
    qaug = qaug_ref[...]
    dn = (((1,), (1,)), ((), ()))
    kaug = jnp.concatenate([kblk, kblk * kblk], axis=1)
    dist = lax.dot_general(qaug, kaug, dn,
                           preferred_element_type=jnp.float32,
                           precision=lax.Precision.HIGHEST)
    bmin = jnp.min(dist, axis=1, keepdims=True)
    ids = lax.broadcasted_iota(jnp.int32, dist.shape, 1) + b * nb
    cand = jnp.where(dist == bmin, ids, _INT_MAX)
    bidx = jnp.min(cand, axis=1, keepdims=True)
    omin_ref[...] = bmin[None]
    oidx_ref[...] = bidx[None]


def _nn_blockmin(queries, keys, nb, interpret=False):
    q, d = queries.shape
    k = keys.shape[0]
    assert k % nb == 0
    nblk = k // nb
    qaug = jnp.concatenate([-2.0 * queries, jnp.ones_like(queries)], axis=1)
    omin, oidx = pl.pallas_call(
        functools.partial(_dist_body, nb=nb),
        grid=(nblk,),
        in_specs=[pl.BlockSpec((q, 2 * d), lambda b: (0, 0)),
                  pl.BlockSpec((nb, d), lambda b: (b, 0))],
        out_specs=[pl.BlockSpec((1, q, 1), lambda b: (b, 0, 0)),
                   pl.BlockSpec((1, q, 1), lambda b: (b, 0, 0))],
        out_shape=[jax.ShapeDtypeStruct((nblk, q, 1), jnp.float32),
                   jax.ShapeDtypeStruct((nblk, q, 1), jnp.int32)],
        interpret=interpret,
    )(qaug, keys)
    return omin[:, :, 0], oidx[:, :, 0]


def kernel(queries, keys, values):
    q = queries.shape[0]
    nb = 50000 if keys.shape[0] % 50000 == 0 else keys.shape[0]
    omin, oidx = _nn_blockmin(queries, keys, nb)
    bb = jnp.argmin(omin, axis=0)
    best_index = oidx[bb, jnp.arange(q)]
    return jnp.take(values, best_index, axis=0)
